# baseline (speedup 1.0000x reference)
import functools

import jax
import jax.numpy as jnp
from jax import lax
from jax.experimental import pallas as pl
from jax.experimental.pallas import tpu as pltpu
from jax.experimental.pallas import tpu_sc as plsc

B, L, H, VOCAB = 128, 512, 768, 8
EPS = 1e-5

NC, NS, LANES = 2, 16, 16
NW = NC * NS
N = B * L
PW = N // NW
CHUNK = 64
NCH = PW // CHUNK


def _table_kernel(tok_ref, pos_ref, gamma_ref, beta_ref, t_ref):
    tok = tok_ref[...]
    pos = pos_ref[...]
    inv_h = 1.0 / H
    ones_row = jnp.ones((1, H), dtype=jnp.float32)
    mp = jnp.mean(pos, axis=1, keepdims=True)
    ep2 = jnp.mean(pos * pos, axis=1, keepdims=True)
    mt = lax.dot_general(ones_row, tok, (((1,), (1,)), ((), ())),
                         preferred_element_type=jnp.float32) * inv_h
    et2 = lax.dot_general(ones_row, tok * tok, (((1,), (1,)), ((), ())),
                          preferred_element_type=jnp.float32) * inv_h
    cross = lax.dot_general(pos, tok, (((1,), (1,)), ((), ())),
                            preferred_element_type=jnp.float32) * inv_h
    mu = mp + mt
    var = ep2 + et2 + 2.0 * cross - mu * mu
    rstd = lax.rsqrt(var + EPS)
    rm = rstd * mu
    gamma = gamma_ref[0]
    beta = beta_ref[0]
    for v in range(VOCAB):
        t = (pos + tok[v, :][None, :]) * rstd[:, v:v + 1] - rm[:, v:v + 1]
        t_ref[:, v, :] = t * gamma[None, :] + beta[None, :]


def _sc_gather(t_hbm, ids_hbm, out_hbm, ids_v, idx_v, rows0, rows1, sg0, sg1, ss0, ss1):
    wid = lax.axis_index("s") * NC + lax.axis_index("c")
    base = wid * PW
    pltpu.sync_copy(ids_hbm.at[pl.ds(base, PW)], ids_v)

    def body(j, _):
        k0 = j * LANES
        ids16 = ids_v[pl.ds(k0, LANES)]
        l = (k0 + lax.iota(jnp.int32, LANES)) & (L - 1)
        idx_v[pl.ds(k0, LANES)] = ids16 + l * VOCAB
        return _

    lax.fori_loop(0, PW // LANES, body, None)

    rows = (rows0, rows1)
    gsems = (sg0, sg1)
    ssems = (ss0, ss1)
    stores = [None, None]
    for g in range(NCH):
        bsel = g % 2
        if stores[bsel] is not None:
            stores[bsel].wait()
        pltpu.async_copy(
            t_hbm.at[idx_v.at[pl.ds(g * CHUNK, CHUNK)]], rows[bsel], gsems[bsel]
        ).wait()
        stores[bsel] = pltpu.async_copy(
            rows[bsel], out_hbm.at[pl.ds(base + g * CHUNK, CHUNK)], ssems[bsel]
        )
    for h in stores:
        h.wait()


def kernel(input_ids, token_table, pos_table, gamma, beta):
    table = pl.pallas_call(
        _table_kernel,
        out_shape=jax.ShapeDtypeStruct((L, VOCAB, H), jnp.float32),
    )(token_table, pos_table, gamma.reshape(1, H), beta.reshape(1, H))
    table = table.reshape(L * VOCAB, H)

    ids_flat = input_ids.reshape(N).astype(jnp.int32)
    sc_call = functools.partial(
        pl.kernel,
        mesh=plsc.VectorSubcoreMesh(core_axis_name="c", subcore_axis_name="s"),
        out_type=jax.ShapeDtypeStruct((N, H), jnp.float32),
        scratch_types=[
            pltpu.VMEM((PW,), jnp.int32),
            pltpu.VMEM((PW,), jnp.int32),
            pltpu.VMEM((CHUNK, H), jnp.float32),
            pltpu.VMEM((CHUNK, H), jnp.float32),
            pltpu.SemaphoreType.DMA,
            pltpu.SemaphoreType.DMA,
            pltpu.SemaphoreType.DMA,
            pltpu.SemaphoreType.DMA,
        ],
    )(_sc_gather)
    out = sc_call(table, ids_flat)
    return out.reshape(B, L, H)


# speedup vs baseline: 1.0023x; 1.0023x over previous
import functools

import jax
import jax.numpy as jnp
from jax import lax
from jax.experimental import pallas as pl
from jax.experimental.pallas import tpu as pltpu
from jax.experimental.pallas import tpu_sc as plsc

B, L, H, VOCAB = 128, 512, 768, 8
EPS = 1e-5

NC, NS, LANES = 2, 16, 16
NW = NC * NS
N = B * L
PW = N // NW
CHUNK = 64
NCH = PW // CHUNK


def _table_kernel(tok_ref, pos_ref, gamma_ref, beta_ref, t_ref):
    tok = tok_ref[...]
    pos = pos_ref[...]
    inv_h = 1.0 / H
    ones_row = jnp.ones((1, H), dtype=jnp.float32)
    mp = jnp.mean(pos, axis=1, keepdims=True)
    ep2 = jnp.mean(pos * pos, axis=1, keepdims=True)
    mt = lax.dot_general(ones_row, tok, (((1,), (1,)), ((), ())),
                         preferred_element_type=jnp.float32) * inv_h
    et2 = lax.dot_general(ones_row, tok * tok, (((1,), (1,)), ((), ())),
                          preferred_element_type=jnp.float32) * inv_h
    cross = lax.dot_general(pos, tok, (((1,), (1,)), ((), ())),
                            preferred_element_type=jnp.float32) * inv_h
    mu = mp + mt
    var = ep2 + et2 + 2.0 * cross - mu * mu
    rstd = lax.rsqrt(var + EPS)
    rm = rstd * mu
    gamma = gamma_ref[0]
    beta = beta_ref[0]
    for v in range(VOCAB):
        t = (pos + tok[v, :][None, :]) * rstd[:, v:v + 1] - rm[:, v:v + 1]
        t_ref[:, v, :] = t * gamma[None, :] + beta[None, :]


def _sc_gather(t_hbm, ids_hbm, out_hbm, ids_v, idx_v, rows0, rows1, sg0, sg1, ss0, ss1):
    wid = lax.axis_index("s") * NC + lax.axis_index("c")
    base = wid * PW
    pltpu.sync_copy(ids_hbm.at[pl.ds(base, PW)], ids_v)

    def body(j, _):
        k0 = j * LANES
        ids16 = ids_v[pl.ds(k0, LANES)]
        l = (k0 + lax.iota(jnp.int32, LANES)) & (L - 1)
        idx_v[pl.ds(k0, LANES)] = ids16 + l * VOCAB
        return _

    lax.fori_loop(0, PW // LANES, body, None)

    rows = (rows0, rows1)
    gsems = (sg0, sg1)
    ssems = (ss0, ss1)

    def gather(g, bsel):
        return pltpu.async_copy(
            t_hbm.at[idx_v.at[pl.ds(g * CHUNK, CHUNK)]], rows[bsel], gsems[bsel]
        )

    stores = [None, None]
    gathers = [gather(0, 0), None]
    for g in range(NCH):
        bsel = g % 2
        gathers[bsel].wait()
        if g + 1 < NCH:
            if stores[1 - bsel] is not None:
                stores[1 - bsel].wait()
            gathers[1 - bsel] = gather(g + 1, 1 - bsel)
        stores[bsel] = pltpu.async_copy(
            rows[bsel], out_hbm.at[pl.ds(base + g * CHUNK, CHUNK)], ssems[bsel]
        )
    for h in stores:
        h.wait()


def kernel(input_ids, token_table, pos_table, gamma, beta):
    table = pl.pallas_call(
        _table_kernel,
        out_shape=jax.ShapeDtypeStruct((L, VOCAB, H), jnp.float32),
    )(token_table, pos_table, gamma.reshape(1, H), beta.reshape(1, H))
    table = table.reshape(L * VOCAB, H)

    ids_flat = input_ids.reshape(N).astype(jnp.int32)
    sc_call = functools.partial(
        pl.kernel,
        mesh=plsc.VectorSubcoreMesh(core_axis_name="c", subcore_axis_name="s"),
        out_type=jax.ShapeDtypeStruct((N, H), jnp.float32),
        scratch_types=[
            pltpu.VMEM((PW,), jnp.int32),
            pltpu.VMEM((PW,), jnp.int32),
            pltpu.VMEM((CHUNK, H), jnp.float32),
            pltpu.VMEM((CHUNK, H), jnp.float32),
            pltpu.SemaphoreType.DMA,
            pltpu.SemaphoreType.DMA,
            pltpu.SemaphoreType.DMA,
            pltpu.SemaphoreType.DMA,
        ],
    )(_sc_gather)
    out = sc_call(table, ids_flat)
    return out.reshape(B, L, H)
